# baseline (speedup 1.0000x reference)
import functools

import jax
import jax.numpy as jnp
from jax import lax
from jax.experimental import pallas as pl
from jax.experimental.pallas import tpu as pltpu
from jax.experimental.pallas import tpu_sc as plsc

N = 10000
E = 320000
H = 128
G = 64
NUM_LAYERS = 2

NC = 2
NS = 16
NW = NC * NS
E_PAD = 327680
EC = E_PAD // NW
EB = 128
NBLK = EC // EB
NBUF = 2
NITER = NBLK // NBUF
NP = 10112
TRASH0 = 10016
RPT = NP // NS
NBLK_F = 160
NBLK_S = 0
ECF = NBLK_F * EB
ECS = NBLK_S * EB

RB = 1000
NRB = N // RB

_sc_mesh = plsc.VectorSubcoreMesh(core_axis_name="c", subcore_axis_name="s")


@functools.partial(
    pl.kernel,
    out_type=[
        jax.ShapeDtypeStruct((E_PAD,), jnp.int32),
        jax.ShapeDtypeStruct((E_PAD,), jnp.int32),
        jax.ShapeDtypeStruct((E_PAD,), jnp.int32),
        jax.ShapeDtypeStruct((NC, NP, H), jnp.float32),
    ],
    mesh=_sc_mesh,
    scratch_types=[
        [pltpu.VMEM((EB,), jnp.int32) for _ in range(NBUF)],
        [pltpu.VMEM((EB,), jnp.int32) for _ in range(NBUF)],
        [pltpu.VMEM((EB,), jnp.int32) for _ in range(NBUF)],
        [pltpu.VMEM((EB, H), jnp.float32) for _ in range(NBUF)],
        pltpu.VMEM((EB,), jnp.int32),
        pltpu.VMEM_SHARED((NP, H), jnp.float32),
        [pltpu.SemaphoreType.DMA for _ in range(NBUF)],
        [pltpu.SemaphoreType.DMA for _ in range(NBUF)],
    ],
)
def _phase_a(dst_hbm, ki_hbm, lut_hbm, zrow_hbm,
             d1_hbm, d2_hbm, d3_hbm, deg_hbm,
             dst_v, ki_v, gix_v, rows_v, dk_v, acc_sh, semg, sems):
    cid = lax.axis_index("c")
    sid = lax.axis_index("s")
    wid = sid * NC + cid
    pltpu.sync_copy(zrow_hbm, acc_sh.at[pl.ds(sid * RPT, RPT)])
    plsc.subcore_barrier()

    trash = jnp.full((16,), TRASH0, jnp.int32) + wid
    lane4 = lax.iota(jnp.int32, 16) * 4

    def body(t, carry):
        for b in range(NBUF):
            @pl.when(t > 0)
            def _():
                pltpu.make_async_copy(rows_v[b], acc_sh.at[dst_v[b]],
                                      sems[b]).wait()
            base = wid * EC + (t * NBUF + b) * EB
            pltpu.sync_copy(dst_hbm.at[pl.ds(base, EB)], dst_v[b])
            pltpu.sync_copy(ki_hbm.at[pl.ds(base, EB)], ki_v[b])
            for j in range(EB // 16):
                sl = pl.ds(j * 16, 16)
                gix_v[b][sl] = lane4 + (64 * j) + ki_v[b][sl]
            pltpu.async_copy(lut_hbm.at[gix_v[b]], rows_v[b], semg[b])
        for b in range(NBUF):
            base = wid * EC + (t * NBUF + b) * EB
            for k, dk_hbm in ((1, d1_hbm), (2, d2_hbm), (3, d3_hbm)):
                for j in range(EB // 16):
                    sl = pl.ds(j * 16, 16)
                    dk_v[sl] = jnp.where(ki_v[b][sl] == k, dst_v[b][sl], trash)
                pltpu.sync_copy(dk_v, dk_hbm.at[pl.ds(base, EB)])
            pltpu.make_async_copy(lut_hbm.at[gix_v[b]], rows_v[b],
                                  semg[b]).wait()
            pltpu.async_copy(rows_v[b], acc_sh.at[dst_v[b]], sems[b], add=True)
        return carry

    lax.fori_loop(0, NITER, body, 0)
    for b in range(NBUF):
        pltpu.make_async_copy(rows_v[b], acc_sh.at[dst_v[b]], sems[b]).wait()
    plsc.subcore_barrier()
    rows = pl.ds(sid * RPT, RPT)
    pltpu.sync_copy(acc_sh.at[rows], deg_hbm.at[cid, rows])


@functools.partial(
    pl.kernel,
    out_type=jax.ShapeDtypeStruct((NC, NP, H), jnp.float32),
    mesh=_sc_mesh,
    scratch_types=(
        [pltpu.VMEM((EB,), jnp.int32) for _ in range(2 * NBUF)] +
        [pltpu.VMEM((EB, H), jnp.float32) for _ in range(NBUF)] +
        [pltpu.VMEM_SHARED((NP, H), jnp.float32)] +
        [pltpu.SemaphoreType.DMA for _ in range(2 * NBUF)]
    ),
)
def _conv_scatter(hs_hbm, src_hbm, dstk_hbm, zrow_hbm, out_hbm, *sc):
    src_v = sc[0:NBUF]
    dst_v = sc[NBUF:2 * NBUF]
    rows_v = sc[2 * NBUF:3 * NBUF]
    acc_sh = sc[3 * NBUF]
    semg = sc[3 * NBUF + 1:3 * NBUF + 1 + NBUF]
    sems = sc[3 * NBUF + 1 + NBUF:]
    cid = lax.axis_index("c")
    sid = lax.axis_index("s")
    wid = sid * NC + cid
    pltpu.sync_copy(zrow_hbm, acc_sh.at[pl.ds(sid * RPT, RPT)])
    plsc.subcore_barrier()

    ebase = jnp.where(cid == 0, sid * ECF, NS * ECF + sid * ECS)
    n_iter = jnp.where(cid == 0, NBLK_F // NBUF, NBLK_S // NBUF)

    def body(t, carry):
        for b in range(NBUF):
            @pl.when(t > 0)
            def _():
                pltpu.make_async_copy(rows_v[b], acc_sh.at[dst_v[b]],
                                      sems[b]).wait()
            base = ebase + (t * NBUF + b) * EB
            pltpu.sync_copy(src_hbm.at[pl.ds(base, EB)], src_v[b])
            pltpu.sync_copy(dstk_hbm.at[pl.ds(base, EB)], dst_v[b])
            pltpu.async_copy(hs_hbm.at[src_v[b]], rows_v[b], semg[b])
        for b in range(NBUF):
            pltpu.make_async_copy(hs_hbm.at[src_v[b]], rows_v[b],
                                  semg[b]).wait()
            pltpu.async_copy(rows_v[b], acc_sh.at[dst_v[b]], sems[b], add=True)
        return carry

    lax.fori_loop(0, n_iter, body, 0)

    @pl.when(n_iter > 0)
    def _():
        for b in range(NBUF):
            pltpu.make_async_copy(rows_v[b], acc_sh.at[dst_v[b]],
                                  sems[b]).wait()

    plsc.subcore_barrier()
    rows = pl.ds(sid * RPT, RPT)
    pltpu.sync_copy(acc_sh.at[rows], out_hbm.at[cid, rows])


def _embed_kernel(x_ref, w_ref, b_ref, o_ref):
    h = lax.dot_general(x_ref[:], w_ref[:], (((1,), (1,)), ((), ())),
                        preferred_element_type=jnp.float32)
    o_ref[:] = h + b_ref[:]


_embed = pl.pallas_call(
    _embed_kernel,
    grid=(NRB,),
    in_specs=[
        pl.BlockSpec((RB, H), lambda i: (i, 0)),
        pl.BlockSpec((H, H), lambda i: (0, 0)),
        pl.BlockSpec((1, H), lambda i: (0, 0)),
    ],
    out_specs=pl.BlockSpec((RB, H), lambda i: (i, 0)),
    out_shape=jax.ShapeDtypeStruct((N, H), jnp.float32),
)


def _make_pre(hop):
    col = 32 * (hop - 1)

    def kern(x_ref, w_ref, g0_ref, g1_ref, o_ref):
        deg = g0_ref[:, col:col + 1] + g1_ref[:, col:col + 1] + 1.0
        dis = lax.rsqrt(deg)
        h = lax.dot_general(x_ref[:], w_ref[:], (((1,), (1,)), ((), ())),
                            preferred_element_type=jnp.float32)
        o_ref[:] = h * dis

    return pl.pallas_call(
        kern,
        grid=(NRB,),
        in_specs=[
            pl.BlockSpec((RB, H), lambda i: (i, 0)),
            pl.BlockSpec((H, H), lambda i: (0, 0)),
            pl.BlockSpec((RB, H), lambda i: (i, 0)),
            pl.BlockSpec((RB, H), lambda i: (i, 0)),
        ],
        out_specs=pl.BlockSpec((RB, H), lambda i: (i, 0)),
        out_shape=jax.ShapeDtypeStruct((N, H), jnp.float32),
    )


_pre = {k: _make_pre(k) for k in (1, 2, 3)}


def _make_post(ks):
    nconv = len(ks)
    cols = [32 * (k - 1) for k in ks]

    def kern(*refs):
        o_ref = refs[-1]
        acc = None
        for c in range(nconv):
            p0, p1, hs, g0, g1, b = refs[6 * c:6 * c + 6]
            cl = cols[c]
            deg = g0[:, cl:cl + 1] + g1[:, cl:cl + 1] + 1.0
            dis = lax.rsqrt(deg)
            t = (dis * (p0[:] + p1[:] + hs[:]) + b[:]) * (1.0 / ks[c])
            acc = t if acc is None else acc + t
        o_ref[:] = jnp.maximum(acc, 0.0)

    in_specs = []
    for _ in range(nconv):
        in_specs += [
            pl.BlockSpec((RB, H), lambda i: (i, 0)),
            pl.BlockSpec((RB, H), lambda i: (i, 0)),
            pl.BlockSpec((RB, H), lambda i: (i, 0)),
            pl.BlockSpec((RB, H), lambda i: (i, 0)),
            pl.BlockSpec((RB, H), lambda i: (i, 0)),
            pl.BlockSpec((1, H), lambda i: (0, 0)),
        ]
    return pl.pallas_call(
        kern,
        grid=(NRB,),
        in_specs=in_specs,
        out_specs=pl.BlockSpec((RB, H), lambda i: (i, 0)),
        out_shape=jax.ShapeDtypeStruct((N, H), jnp.float32),
    )


_post = {n: _make_post(tuple(range(1, n + 2))) for n in range(NUM_LAYERS + 1)}


def _pool_kernel(x_ref, b_ref, add_ref, cnt_ref, mx_ref):
    i = pl.program_id(0)

    @pl.when(i == 0)
    def _():
        add_ref[:] = jnp.zeros_like(add_ref)
        cnt_ref[:] = jnp.zeros_like(cnt_ref)
        mx_ref[:] = jnp.zeros_like(mx_ref)

    xb = x_ref[:]
    bb = b_ref[:]
    oh = (bb == lax.broadcasted_iota(jnp.int32, (RB, G), 1)).astype(jnp.float32)
    add_ref[:] += lax.dot_general(oh, xb, (((0,), (0,)), ((), ())),
                                  preferred_element_type=jnp.float32)
    cnt_ref[:] += jnp.sum(oh, axis=0)[:, None]
    lo = bb[0, 0]
    hi = bb[RB - 1, 0]
    for g in range(G):
        @pl.when(jnp.logical_and(lo <= g, g <= hi))
        def _():
            mg = jnp.max(jnp.where(bb == g, xb, 0.0), axis=0)
            mx_ref[g, :] = jnp.maximum(mx_ref[g, :], mg)


_pool = pl.pallas_call(
    _pool_kernel,
    grid=(NRB,),
    in_specs=[
        pl.BlockSpec((RB, H), lambda i: (i, 0)),
        pl.BlockSpec((RB, 1), lambda i: (i, 0)),
    ],
    out_specs=[
        pl.BlockSpec((G, H), lambda i: (0, 0)),
        pl.BlockSpec((G, H), lambda i: (0, 0)),
        pl.BlockSpec((G, H), lambda i: (0, 0)),
    ],
    out_shape=[
        jax.ShapeDtypeStruct((G, H), jnp.float32),
        jax.ShapeDtypeStruct((G, H), jnp.float32),
        jax.ShapeDtypeStruct((G, H), jnp.float32),
    ],
)


def _readout_kernel(add_ref, cnt_ref, mx_ref, w1_ref, b1_ref, w2_ref, b2_ref,
                    o_ref):
    add = add_ref[:]
    cnt = cnt_ref[:, 0:1]
    mx = mx_ref[:]
    mean = add / jnp.maximum(cnt, 1.0)
    pooled = jnp.concatenate([add, mx, mean], axis=1)
    h1 = lax.dot_general(pooled, w1_ref[:], (((1,), (1,)), ((), ())),
                         preferred_element_type=jnp.float32) + b1_ref[:]
    h1 = jnp.where(h1 > 0, h1, 0.01 * h1)
    o_ref[:] = lax.dot_general(h1, w2_ref[:], (((1,), (1,)), ((), ())),
                               preferred_element_type=jnp.float32) + b2_ref[:]


_readout = pl.pallas_call(
    _readout_kernel,
    out_shape=jax.ShapeDtypeStruct((G, H), jnp.float32),
)


def kernel(x, k_edge_index, k_idx, batch, W_emb, b_emb, conv_W, conv_b,
           r1_W, r1_b, r2_W, r2_b):
    pad = E_PAD - E
    src = jnp.concatenate([k_edge_index[0], jnp.zeros((pad,), jnp.int32)])
    dst = jnp.concatenate([k_edge_index[1], jnp.zeros((pad,), jnp.int32)])
    ki = jnp.concatenate([k_idx, jnp.zeros((pad,), jnp.int32)])
    zrow = jnp.zeros((RPT, H), jnp.float32)
    base_lut = jnp.zeros((4, H), jnp.float32)
    for k in (1, 2, 3):
        base_lut = base_lut.at[k, 32 * (k - 1):32 * k].set(1.0)
    lut = jnp.tile(base_lut, (EB, 1))

    d1, d2, d3, deg = _phase_a(dst, ki, lut, zrow)
    dstks = [d1, d2, d3]
    g0, g1 = deg[0], deg[1]

    h0 = _embed(x, W_emb, b_emb.reshape(1, H))
    xl = [h0]
    ci = 0
    for l in range(NUM_LAYERS + 1):
        parts = []
        for k in range(1, l + 2):
            hs = _pre[k](xl[-k], conv_W[ci], g0, g1)
            scat = _conv_scatter(hs, src, dstks[k - 1], zrow)
            parts += [scat[0], scat[1], hs, g0, g1,
                      conv_b[ci].reshape(1, H)]
            ci += 1
        xl.append(_post[l](*parts))

    add, cnt, mx = _pool(xl[-1], batch.reshape(N, 1))
    return _readout(add, cnt, mx, r1_W, r1_b.reshape(1, -1),
                    r2_W, r2_b.reshape(1, -1))


# speedup vs baseline: 1.4649x; 1.4649x over previous
import functools

import jax
import jax.numpy as jnp
from jax import lax
from jax.experimental import pallas as pl
from jax.experimental.pallas import tpu as pltpu
from jax.experimental.pallas import tpu_sc as plsc

N = 10000
E = 320000
H = 128
G = 64
NUM_LAYERS = 2

NC = 2
NS = 16
NW = NC * NS
E_PAD = 327680
EC = E_PAD // NW
EB = 128
NBLK = EC // EB
NBUF = 2
NITER = NBLK // NBUF
NP = 10112
TRASH0 = 10016
RPT = NP // NS
NBLK_F = 152
NBLK_S = 8
ECF = NBLK_F * EB
ECS = NBLK_S * EB

RB = 1000
NRB = N // RB

_sc_mesh = plsc.VectorSubcoreMesh(core_axis_name="c", subcore_axis_name="s")


@functools.partial(
    pl.kernel,
    out_type=[
        jax.ShapeDtypeStruct((E_PAD,), jnp.int32),
        jax.ShapeDtypeStruct((E_PAD,), jnp.int32),
        jax.ShapeDtypeStruct((E_PAD,), jnp.int32),
        jax.ShapeDtypeStruct((NC, NP, H), jnp.float32),
    ],
    mesh=_sc_mesh,
    scratch_types=[
        [pltpu.VMEM((EB,), jnp.int32) for _ in range(NBUF)],
        [pltpu.VMEM((EB,), jnp.int32) for _ in range(NBUF)],
        [pltpu.VMEM((EB,), jnp.int32) for _ in range(NBUF)],
        [pltpu.VMEM((EB, H), jnp.float32) for _ in range(NBUF)],
        pltpu.VMEM((EB,), jnp.int32),
        pltpu.VMEM_SHARED((NP, H), jnp.float32),
        [pltpu.SemaphoreType.DMA for _ in range(NBUF)],
        [pltpu.SemaphoreType.DMA for _ in range(NBUF)],
    ],
)
def _phase_a(dst_hbm, ki_hbm, lut_hbm, zrow_hbm,
             d1_hbm, d2_hbm, d3_hbm, deg_hbm,
             dst_v, ki_v, gix_v, rows_v, dk_v, acc_sh, semg, sems):
    cid = lax.axis_index("c")
    sid = lax.axis_index("s")
    wid = sid * NC + cid
    pltpu.sync_copy(zrow_hbm, acc_sh.at[pl.ds(sid * RPT, RPT)])
    plsc.subcore_barrier()

    trash = jnp.full((16,), TRASH0, jnp.int32) + wid
    lane4 = lax.iota(jnp.int32, 16) * 4

    def body(t, carry):
        for b in range(NBUF):
            @pl.when(t > 0)
            def _():
                pltpu.make_async_copy(rows_v[b], acc_sh.at[dst_v[b]],
                                      sems[b]).wait()
            base = wid * EC + (t * NBUF + b) * EB
            pltpu.sync_copy(dst_hbm.at[pl.ds(base, EB)], dst_v[b])
            pltpu.sync_copy(ki_hbm.at[pl.ds(base, EB)], ki_v[b])
            for j in range(EB // 16):
                sl = pl.ds(j * 16, 16)
                gix_v[b][sl] = lane4 + (64 * j) + ki_v[b][sl]
            pltpu.async_copy(lut_hbm.at[gix_v[b]], rows_v[b], semg[b])
        for b in range(NBUF):
            base = wid * EC + (t * NBUF + b) * EB
            for k, dk_hbm in ((1, d1_hbm), (2, d2_hbm), (3, d3_hbm)):
                for j in range(EB // 16):
                    sl = pl.ds(j * 16, 16)
                    dk_v[sl] = jnp.where(ki_v[b][sl] == k, dst_v[b][sl], trash)
                pltpu.sync_copy(dk_v, dk_hbm.at[pl.ds(base, EB)])
            pltpu.make_async_copy(lut_hbm.at[gix_v[b]], rows_v[b],
                                  semg[b]).wait()
            pltpu.async_copy(rows_v[b], acc_sh.at[dst_v[b]], sems[b], add=True)
        return carry

    lax.fori_loop(0, NITER, body, 0)
    for b in range(NBUF):
        pltpu.make_async_copy(rows_v[b], acc_sh.at[dst_v[b]], sems[b]).wait()
    plsc.subcore_barrier()
    rows = pl.ds(sid * RPT, RPT)
    pltpu.sync_copy(acc_sh.at[rows], deg_hbm.at[cid, rows])


@functools.partial(
    pl.kernel,
    out_type=jax.ShapeDtypeStruct((NC, NP, H), jnp.float32),
    mesh=_sc_mesh,
    scratch_types=(
        [pltpu.VMEM((EB,), jnp.int32) for _ in range(2 * NBUF)] +
        [pltpu.VMEM((EB, H), jnp.float32) for _ in range(NBUF)] +
        [pltpu.VMEM_SHARED((NP, H), jnp.float32)] +
        [pltpu.SemaphoreType.DMA for _ in range(2 * NBUF)]
    ),
)
def _conv_scatter(hs_hbm, src_hbm, dstk_hbm, zrow_hbm, out_hbm, *sc):
    src_v = sc[0:NBUF]
    dst_v = sc[NBUF:2 * NBUF]
    rows_v = sc[2 * NBUF:3 * NBUF]
    acc_sh = sc[3 * NBUF]
    semg = sc[3 * NBUF + 1:3 * NBUF + 1 + NBUF]
    sems = sc[3 * NBUF + 1 + NBUF:]
    cid = lax.axis_index("c")
    sid = lax.axis_index("s")
    wid = sid * NC + cid
    pltpu.sync_copy(zrow_hbm, acc_sh.at[pl.ds(sid * RPT, RPT)])
    plsc.subcore_barrier()

    ebase = jnp.where(cid == 0, sid * ECF, NS * ECF + sid * ECS)
    n_iter = jnp.where(cid == 0, NBLK_F // NBUF, NBLK_S // NBUF)

    def body(t, carry):
        for b in range(NBUF):
            @pl.when(t > 0)
            def _():
                pltpu.make_async_copy(rows_v[b], acc_sh.at[dst_v[b]],
                                      sems[b]).wait()
            base = ebase + (t * NBUF + b) * EB
            pltpu.sync_copy(src_hbm.at[pl.ds(base, EB)], src_v[b])
            pltpu.sync_copy(dstk_hbm.at[pl.ds(base, EB)], dst_v[b])
            pltpu.async_copy(hs_hbm.at[src_v[b]], rows_v[b], semg[b])
        for b in range(NBUF):
            pltpu.make_async_copy(hs_hbm.at[src_v[b]], rows_v[b],
                                  semg[b]).wait()
            pltpu.async_copy(rows_v[b], acc_sh.at[dst_v[b]], sems[b], add=True)
        return carry

    lax.fori_loop(0, n_iter, body, 0)

    @pl.when(n_iter > 0)
    def _():
        for b in range(NBUF):
            pltpu.make_async_copy(rows_v[b], acc_sh.at[dst_v[b]],
                                  sems[b]).wait()

    plsc.subcore_barrier()
    rows = pl.ds(sid * RPT, RPT)
    pltpu.sync_copy(acc_sh.at[rows], out_hbm.at[cid, rows])


def _embed_kernel(x_ref, w_ref, b_ref, o_ref):
    h = lax.dot_general(x_ref[:], w_ref[:], (((1,), (1,)), ((), ())),
                        preferred_element_type=jnp.float32)
    o_ref[:] = h + b_ref[:]


_embed = pl.pallas_call(
    _embed_kernel,
    grid=(NRB,),
    in_specs=[
        pl.BlockSpec((RB, H), lambda i: (i, 0)),
        pl.BlockSpec((H, H), lambda i: (0, 0)),
        pl.BlockSpec((1, H), lambda i: (0, 0)),
    ],
    out_specs=pl.BlockSpec((RB, H), lambda i: (i, 0)),
    out_shape=jax.ShapeDtypeStruct((N, H), jnp.float32),
)


def _make_pre(hop):
    col = 32 * (hop - 1)

    def kern(x_ref, w_ref, g0_ref, g1_ref, o_ref):
        deg = g0_ref[:, col:col + 1] + g1_ref[:, col:col + 1] + 1.0
        dis = lax.rsqrt(deg)
        h = lax.dot_general(x_ref[:], w_ref[:], (((1,), (1,)), ((), ())),
                            preferred_element_type=jnp.float32)
        o_ref[:] = h * dis

    return pl.pallas_call(
        kern,
        grid=(NRB,),
        in_specs=[
            pl.BlockSpec((RB, H), lambda i: (i, 0)),
            pl.BlockSpec((H, H), lambda i: (0, 0)),
            pl.BlockSpec((RB, H), lambda i: (i, 0)),
            pl.BlockSpec((RB, H), lambda i: (i, 0)),
        ],
        out_specs=pl.BlockSpec((RB, H), lambda i: (i, 0)),
        out_shape=jax.ShapeDtypeStruct((N, H), jnp.float32),
    )


_pre = {k: _make_pre(k) for k in (1, 2, 3)}


def _make_post(ks):
    nconv = len(ks)
    cols = [32 * (k - 1) for k in ks]

    def kern(*refs):
        o_ref = refs[-1]
        acc = None
        for c in range(nconv):
            p0, p1, hs, g0, g1, b = refs[6 * c:6 * c + 6]
            cl = cols[c]
            deg = g0[:, cl:cl + 1] + g1[:, cl:cl + 1] + 1.0
            dis = lax.rsqrt(deg)
            t = (dis * (p0[:] + p1[:] + hs[:]) + b[:]) * (1.0 / ks[c])
            acc = t if acc is None else acc + t
        o_ref[:] = jnp.maximum(acc, 0.0)

    in_specs = []
    for _ in range(nconv):
        in_specs += [
            pl.BlockSpec((RB, H), lambda i: (i, 0)),
            pl.BlockSpec((RB, H), lambda i: (i, 0)),
            pl.BlockSpec((RB, H), lambda i: (i, 0)),
            pl.BlockSpec((RB, H), lambda i: (i, 0)),
            pl.BlockSpec((RB, H), lambda i: (i, 0)),
            pl.BlockSpec((1, H), lambda i: (0, 0)),
        ]
    return pl.pallas_call(
        kern,
        grid=(NRB,),
        in_specs=in_specs,
        out_specs=pl.BlockSpec((RB, H), lambda i: (i, 0)),
        out_shape=jax.ShapeDtypeStruct((N, H), jnp.float32),
    )


_post = {n: _make_post(tuple(range(1, n + 2))) for n in range(NUM_LAYERS + 1)}


def _pool_kernel(x_ref, b_ref, add_ref, cnt_ref, mx_ref):
    i = pl.program_id(0)

    @pl.when(i == 0)
    def _():
        add_ref[:] = jnp.zeros_like(add_ref)
        cnt_ref[:] = jnp.zeros_like(cnt_ref)
        mx_ref[:] = jnp.zeros_like(mx_ref)

    xb = x_ref[:]
    bb = b_ref[:]
    oh = (bb == lax.broadcasted_iota(jnp.int32, (RB, G), 1)).astype(jnp.float32)
    add_ref[:] += lax.dot_general(oh, xb, (((0,), (0,)), ((), ())),
                                  preferred_element_type=jnp.float32)
    cnt_ref[:] += jnp.sum(oh, axis=0)[:, None]
    lo = bb[0, 0]
    hi = bb[RB - 1, 0]
    for g in range(G):
        @pl.when(jnp.logical_and(lo <= g, g <= hi))
        def _():
            mg = jnp.max(jnp.where(bb == g, xb, 0.0), axis=0)
            mx_ref[g, :] = jnp.maximum(mx_ref[g, :], mg)


_pool = pl.pallas_call(
    _pool_kernel,
    grid=(NRB,),
    in_specs=[
        pl.BlockSpec((RB, H), lambda i: (i, 0)),
        pl.BlockSpec((RB, 1), lambda i: (i, 0)),
    ],
    out_specs=[
        pl.BlockSpec((G, H), lambda i: (0, 0)),
        pl.BlockSpec((G, H), lambda i: (0, 0)),
        pl.BlockSpec((G, H), lambda i: (0, 0)),
    ],
    out_shape=[
        jax.ShapeDtypeStruct((G, H), jnp.float32),
        jax.ShapeDtypeStruct((G, H), jnp.float32),
        jax.ShapeDtypeStruct((G, H), jnp.float32),
    ],
)


def _readout_kernel(add_ref, cnt_ref, mx_ref, w1_ref, b1_ref, w2_ref, b2_ref,
                    o_ref):
    add = add_ref[:]
    cnt = cnt_ref[:, 0:1]
    mx = mx_ref[:]
    mean = add / jnp.maximum(cnt, 1.0)
    pooled = jnp.concatenate([add, mx, mean], axis=1)
    h1 = lax.dot_general(pooled, w1_ref[:], (((1,), (1,)), ((), ())),
                         preferred_element_type=jnp.float32) + b1_ref[:]
    h1 = jnp.where(h1 > 0, h1, 0.01 * h1)
    o_ref[:] = lax.dot_general(h1, w2_ref[:], (((1,), (1,)), ((), ())),
                               preferred_element_type=jnp.float32) + b2_ref[:]


_readout = pl.pallas_call(
    _readout_kernel,
    out_shape=jax.ShapeDtypeStruct((G, H), jnp.float32),
)


def kernel(x, k_edge_index, k_idx, batch, W_emb, b_emb, conv_W, conv_b,
           r1_W, r1_b, r2_W, r2_b):
    pad = E_PAD - E
    src = jnp.concatenate([k_edge_index[0], jnp.zeros((pad,), jnp.int32)])
    dst = jnp.concatenate([k_edge_index[1], jnp.zeros((pad,), jnp.int32)])
    ki = jnp.concatenate([k_idx, jnp.zeros((pad,), jnp.int32)])
    zrow = jnp.zeros((RPT, H), jnp.float32)
    base_lut = jnp.zeros((4, H), jnp.float32)
    for k in (1, 2, 3):
        base_lut = base_lut.at[k, 32 * (k - 1):32 * k].set(1.0)
    lut = jnp.tile(base_lut, (EB, 1))

    d1, d2, d3, deg = _phase_a(dst, ki, lut, zrow)
    dstks = [d1, d2, d3]
    g0, g1 = deg[0], deg[1]

    h0 = _embed(x, W_emb, b_emb.reshape(1, H))
    xl = [h0]
    ci = 0
    for l in range(NUM_LAYERS + 1):
        parts = []
        for k in range(1, l + 2):
            hs = _pre[k](xl[-k], conv_W[ci], g0, g1)
            scat = _conv_scatter(hs, src, dstks[k - 1], zrow)
            parts += [scat[0], scat[1], hs, g0, g1,
                      conv_b[ci].reshape(1, H)]
            ci += 1
        xl.append(_post[l](*parts))

    add, cnt, mx = _pool(xl[-1], batch.reshape(N, 1))
    return _readout(add, cnt, mx, r1_W, r1_b.reshape(1, -1),
                    r2_W, r2_b.reshape(1, -1))


# speedup vs baseline: 1.4652x; 1.0002x over previous
import functools

import jax
import jax.numpy as jnp
from jax import lax
from jax.experimental import pallas as pl
from jax.experimental.pallas import tpu as pltpu
from jax.experimental.pallas import tpu_sc as plsc

N = 10000
E = 320000
H = 128
G = 64
NUM_LAYERS = 2

NC = 2
NS = 16
NW = NC * NS
E_PAD = 327680
EC = E_PAD // NW
EB = 128
NBLK = EC // EB
NBUF = 2
NITER = NBLK // NBUF
NP = 10112
TRASH0 = 10016
RPT = NP // NS
NBLK_F = 144
NBLK_S = 16
ECF = NBLK_F * EB
ECS = NBLK_S * EB

RB = 1000
NRB = N // RB

_sc_mesh = plsc.VectorSubcoreMesh(core_axis_name="c", subcore_axis_name="s")


@functools.partial(
    pl.kernel,
    out_type=[
        jax.ShapeDtypeStruct((E_PAD,), jnp.int32),
        jax.ShapeDtypeStruct((E_PAD,), jnp.int32),
        jax.ShapeDtypeStruct((E_PAD,), jnp.int32),
        jax.ShapeDtypeStruct((NC, NP, H), jnp.float32),
    ],
    mesh=_sc_mesh,
    scratch_types=[
        [pltpu.VMEM((EB,), jnp.int32) for _ in range(NBUF)],
        [pltpu.VMEM((EB,), jnp.int32) for _ in range(NBUF)],
        [pltpu.VMEM((EB,), jnp.int32) for _ in range(NBUF)],
        [pltpu.VMEM((EB, H), jnp.float32) for _ in range(NBUF)],
        pltpu.VMEM((EB,), jnp.int32),
        pltpu.VMEM_SHARED((NP, H), jnp.float32),
        [pltpu.SemaphoreType.DMA for _ in range(NBUF)],
        [pltpu.SemaphoreType.DMA for _ in range(NBUF)],
    ],
)
def _phase_a(dst_hbm, ki_hbm, lut_hbm, zrow_hbm,
             d1_hbm, d2_hbm, d3_hbm, deg_hbm,
             dst_v, ki_v, gix_v, rows_v, dk_v, acc_sh, semg, sems):
    cid = lax.axis_index("c")
    sid = lax.axis_index("s")
    wid = sid * NC + cid
    pltpu.sync_copy(zrow_hbm, acc_sh.at[pl.ds(sid * RPT, RPT)])
    plsc.subcore_barrier()

    trash = jnp.full((16,), TRASH0, jnp.int32) + wid
    lane4 = lax.iota(jnp.int32, 16) * 4

    def body(t, carry):
        for b in range(NBUF):
            @pl.when(t > 0)
            def _():
                pltpu.make_async_copy(rows_v[b], acc_sh.at[dst_v[b]],
                                      sems[b]).wait()
            base = wid * EC + (t * NBUF + b) * EB
            pltpu.sync_copy(dst_hbm.at[pl.ds(base, EB)], dst_v[b])
            pltpu.sync_copy(ki_hbm.at[pl.ds(base, EB)], ki_v[b])
            for j in range(EB // 16):
                sl = pl.ds(j * 16, 16)
                gix_v[b][sl] = lane4 + (64 * j) + ki_v[b][sl]
            pltpu.async_copy(lut_hbm.at[gix_v[b]], rows_v[b], semg[b])
        for b in range(NBUF):
            base = wid * EC + (t * NBUF + b) * EB
            for k, dk_hbm in ((1, d1_hbm), (2, d2_hbm), (3, d3_hbm)):
                for j in range(EB // 16):
                    sl = pl.ds(j * 16, 16)
                    dk_v[sl] = jnp.where(ki_v[b][sl] == k, dst_v[b][sl], trash)
                pltpu.sync_copy(dk_v, dk_hbm.at[pl.ds(base, EB)])
            pltpu.make_async_copy(lut_hbm.at[gix_v[b]], rows_v[b],
                                  semg[b]).wait()
            pltpu.async_copy(rows_v[b], acc_sh.at[dst_v[b]], sems[b], add=True)
        return carry

    lax.fori_loop(0, NITER, body, 0)
    for b in range(NBUF):
        pltpu.make_async_copy(rows_v[b], acc_sh.at[dst_v[b]], sems[b]).wait()
    plsc.subcore_barrier()
    rows = pl.ds(sid * RPT, RPT)
    pltpu.sync_copy(acc_sh.at[rows], deg_hbm.at[cid, rows])


@functools.partial(
    pl.kernel,
    out_type=jax.ShapeDtypeStruct((NC, NP, H), jnp.float32),
    mesh=_sc_mesh,
    scratch_types=(
        [pltpu.VMEM((EB,), jnp.int32) for _ in range(2 * NBUF)] +
        [pltpu.VMEM((EB, H), jnp.float32) for _ in range(NBUF)] +
        [pltpu.VMEM_SHARED((NP, H), jnp.float32)] +
        [pltpu.SemaphoreType.DMA for _ in range(2 * NBUF)]
    ),
)
def _conv_scatter(hs_hbm, src_hbm, dstk_hbm, zrow_hbm, out_hbm, *sc):
    src_v = sc[0:NBUF]
    dst_v = sc[NBUF:2 * NBUF]
    rows_v = sc[2 * NBUF:3 * NBUF]
    acc_sh = sc[3 * NBUF]
    semg = sc[3 * NBUF + 1:3 * NBUF + 1 + NBUF]
    sems = sc[3 * NBUF + 1 + NBUF:]
    cid = lax.axis_index("c")
    sid = lax.axis_index("s")
    wid = sid * NC + cid
    pltpu.sync_copy(zrow_hbm, acc_sh.at[pl.ds(sid * RPT, RPT)])
    plsc.subcore_barrier()

    ebase = jnp.where(cid == 0, sid * ECF, NS * ECF + sid * ECS)
    n_iter = jnp.where(cid == 0, NBLK_F // NBUF, NBLK_S // NBUF)

    def body(t, carry):
        for b in range(NBUF):
            @pl.when(t > 0)
            def _():
                pltpu.make_async_copy(rows_v[b], acc_sh.at[dst_v[b]],
                                      sems[b]).wait()
            base = ebase + (t * NBUF + b) * EB
            pltpu.sync_copy(src_hbm.at[pl.ds(base, EB)], src_v[b])
            pltpu.sync_copy(dstk_hbm.at[pl.ds(base, EB)], dst_v[b])
            pltpu.async_copy(hs_hbm.at[src_v[b]], rows_v[b], semg[b])
        for b in range(NBUF):
            pltpu.make_async_copy(hs_hbm.at[src_v[b]], rows_v[b],
                                  semg[b]).wait()
            pltpu.async_copy(rows_v[b], acc_sh.at[dst_v[b]], sems[b], add=True)
        return carry

    lax.fori_loop(0, n_iter, body, 0)

    @pl.when(n_iter > 0)
    def _():
        for b in range(NBUF):
            pltpu.make_async_copy(rows_v[b], acc_sh.at[dst_v[b]],
                                  sems[b]).wait()

    plsc.subcore_barrier()
    rows = pl.ds(sid * RPT, RPT)
    pltpu.sync_copy(acc_sh.at[rows], out_hbm.at[cid, rows])


def _embed_kernel(x_ref, w_ref, b_ref, o_ref):
    h = lax.dot_general(x_ref[:], w_ref[:], (((1,), (1,)), ((), ())),
                        preferred_element_type=jnp.float32)
    o_ref[:] = h + b_ref[:]


_embed = pl.pallas_call(
    _embed_kernel,
    grid=(NRB,),
    in_specs=[
        pl.BlockSpec((RB, H), lambda i: (i, 0)),
        pl.BlockSpec((H, H), lambda i: (0, 0)),
        pl.BlockSpec((1, H), lambda i: (0, 0)),
    ],
    out_specs=pl.BlockSpec((RB, H), lambda i: (i, 0)),
    out_shape=jax.ShapeDtypeStruct((N, H), jnp.float32),
)


def _make_pre(hop):
    col = 32 * (hop - 1)

    def kern(x_ref, w_ref, g0_ref, g1_ref, o_ref):
        deg = g0_ref[:, col:col + 1] + g1_ref[:, col:col + 1] + 1.0
        dis = lax.rsqrt(deg)
        h = lax.dot_general(x_ref[:], w_ref[:], (((1,), (1,)), ((), ())),
                            preferred_element_type=jnp.float32)
        o_ref[:] = h * dis

    return pl.pallas_call(
        kern,
        grid=(NRB,),
        in_specs=[
            pl.BlockSpec((RB, H), lambda i: (i, 0)),
            pl.BlockSpec((H, H), lambda i: (0, 0)),
            pl.BlockSpec((RB, H), lambda i: (i, 0)),
            pl.BlockSpec((RB, H), lambda i: (i, 0)),
        ],
        out_specs=pl.BlockSpec((RB, H), lambda i: (i, 0)),
        out_shape=jax.ShapeDtypeStruct((N, H), jnp.float32),
    )


_pre = {k: _make_pre(k) for k in (1, 2, 3)}


def _make_post(ks):
    nconv = len(ks)
    cols = [32 * (k - 1) for k in ks]

    def kern(*refs):
        o_ref = refs[-1]
        acc = None
        for c in range(nconv):
            p0, p1, hs, g0, g1, b = refs[6 * c:6 * c + 6]
            cl = cols[c]
            deg = g0[:, cl:cl + 1] + g1[:, cl:cl + 1] + 1.0
            dis = lax.rsqrt(deg)
            t = (dis * (p0[:] + p1[:] + hs[:]) + b[:]) * (1.0 / ks[c])
            acc = t if acc is None else acc + t
        o_ref[:] = jnp.maximum(acc, 0.0)

    in_specs = []
    for _ in range(nconv):
        in_specs += [
            pl.BlockSpec((RB, H), lambda i: (i, 0)),
            pl.BlockSpec((RB, H), lambda i: (i, 0)),
            pl.BlockSpec((RB, H), lambda i: (i, 0)),
            pl.BlockSpec((RB, H), lambda i: (i, 0)),
            pl.BlockSpec((RB, H), lambda i: (i, 0)),
            pl.BlockSpec((1, H), lambda i: (0, 0)),
        ]
    return pl.pallas_call(
        kern,
        grid=(NRB,),
        in_specs=in_specs,
        out_specs=pl.BlockSpec((RB, H), lambda i: (i, 0)),
        out_shape=jax.ShapeDtypeStruct((N, H), jnp.float32),
    )


_post = {n: _make_post(tuple(range(1, n + 2))) for n in range(NUM_LAYERS + 1)}


def _pool_kernel(x_ref, b_ref, add_ref, cnt_ref, mx_ref):
    i = pl.program_id(0)

    @pl.when(i == 0)
    def _():
        add_ref[:] = jnp.zeros_like(add_ref)
        cnt_ref[:] = jnp.zeros_like(cnt_ref)
        mx_ref[:] = jnp.zeros_like(mx_ref)

    xb = x_ref[:]
    bb = b_ref[:]
    oh = (bb == lax.broadcasted_iota(jnp.int32, (RB, G), 1)).astype(jnp.float32)
    add_ref[:] += lax.dot_general(oh, xb, (((0,), (0,)), ((), ())),
                                  preferred_element_type=jnp.float32)
    cnt_ref[:] += jnp.sum(oh, axis=0)[:, None]
    lo = bb[0, 0]
    hi = bb[RB - 1, 0]
    for g in range(G):
        @pl.when(jnp.logical_and(lo <= g, g <= hi))
        def _():
            mg = jnp.max(jnp.where(bb == g, xb, 0.0), axis=0)
            mx_ref[g, :] = jnp.maximum(mx_ref[g, :], mg)


_pool = pl.pallas_call(
    _pool_kernel,
    grid=(NRB,),
    in_specs=[
        pl.BlockSpec((RB, H), lambda i: (i, 0)),
        pl.BlockSpec((RB, 1), lambda i: (i, 0)),
    ],
    out_specs=[
        pl.BlockSpec((G, H), lambda i: (0, 0)),
        pl.BlockSpec((G, H), lambda i: (0, 0)),
        pl.BlockSpec((G, H), lambda i: (0, 0)),
    ],
    out_shape=[
        jax.ShapeDtypeStruct((G, H), jnp.float32),
        jax.ShapeDtypeStruct((G, H), jnp.float32),
        jax.ShapeDtypeStruct((G, H), jnp.float32),
    ],
)


def _readout_kernel(add_ref, cnt_ref, mx_ref, w1_ref, b1_ref, w2_ref, b2_ref,
                    o_ref):
    add = add_ref[:]
    cnt = cnt_ref[:, 0:1]
    mx = mx_ref[:]
    mean = add / jnp.maximum(cnt, 1.0)
    pooled = jnp.concatenate([add, mx, mean], axis=1)
    h1 = lax.dot_general(pooled, w1_ref[:], (((1,), (1,)), ((), ())),
                         preferred_element_type=jnp.float32) + b1_ref[:]
    h1 = jnp.where(h1 > 0, h1, 0.01 * h1)
    o_ref[:] = lax.dot_general(h1, w2_ref[:], (((1,), (1,)), ((), ())),
                               preferred_element_type=jnp.float32) + b2_ref[:]


_readout = pl.pallas_call(
    _readout_kernel,
    out_shape=jax.ShapeDtypeStruct((G, H), jnp.float32),
)


def kernel(x, k_edge_index, k_idx, batch, W_emb, b_emb, conv_W, conv_b,
           r1_W, r1_b, r2_W, r2_b):
    pad = E_PAD - E
    src = jnp.concatenate([k_edge_index[0], jnp.zeros((pad,), jnp.int32)])
    dst = jnp.concatenate([k_edge_index[1], jnp.zeros((pad,), jnp.int32)])
    ki = jnp.concatenate([k_idx, jnp.zeros((pad,), jnp.int32)])
    zrow = jnp.zeros((RPT, H), jnp.float32)
    base_lut = jnp.zeros((4, H), jnp.float32)
    for k in (1, 2, 3):
        base_lut = base_lut.at[k, 32 * (k - 1):32 * k].set(1.0)
    lut = jnp.tile(base_lut, (EB, 1))

    d1, d2, d3, deg = _phase_a(dst, ki, lut, zrow)
    dstks = [d1, d2, d3]
    g0, g1 = deg[0], deg[1]

    h0 = _embed(x, W_emb, b_emb.reshape(1, H))
    xl = [h0]
    ci = 0
    for l in range(NUM_LAYERS + 1):
        parts = []
        for k in range(1, l + 2):
            hs = _pre[k](xl[-k], conv_W[ci], g0, g1)
            scat = _conv_scatter(hs, src, dstks[k - 1], zrow)
            parts += [scat[0], scat[1], hs, g0, g1,
                      conv_b[ci].reshape(1, H)]
            ci += 1
        xl.append(_post[l](*parts))

    add, cnt, mx = _pool(xl[-1], batch.reshape(N, 1))
    return _readout(add, cnt, mx, r1_W, r1_b.reshape(1, -1),
                    r2_W, r2_b.reshape(1, -1))
